# baseline (speedup 1.0000x reference)
import functools

import jax
import jax.numpy as jnp
from jax import lax
from jax.experimental import pallas as pl
from jax.experimental.pallas import tpu as pltpu
from jax.experimental.pallas import tpu_sc as plsc

_NUM_CLASSES = 1000000
_DROPOUT_PROB = 0.1

_NC = 2
_NS = 16
_NW = _NC * _NS
_CHUNK = 128


@functools.lru_cache(maxsize=None)
def _make_gather(vocab: int, d: int, b: int):
    b_per_w = b // _NW
    n_chunks = b_per_w // _CHUNK
    mesh = plsc.VectorSubcoreMesh(core_axis_name="c", subcore_axis_name="s")

    @functools.partial(
        pl.kernel,
        out_type=jax.ShapeDtypeStruct((b, d), jnp.float32),
        mesh=mesh,
        scratch_types=[
            pltpu.VMEM((n_chunks, _CHUNK), jnp.int32),
            pltpu.VMEM((b_per_w, d), jnp.float32),
            pltpu.SemaphoreType.DMA,
        ],
        compiler_params=pltpu.CompilerParams(use_tc_tiling_on_sc=False),
    )
    def gather_kernel(idx_hbm, table_hbm, out_hbm, idx_v, rows_v, sem):
        wid = lax.axis_index("s") * _NC + lax.axis_index("c")
        base = wid * b_per_w
        pltpu.sync_copy(idx_hbm.at[wid], idx_v)
        copies = [
            pltpu.async_copy(
                table_hbm.at[idx_v.at[j]],
                rows_v.at[pl.ds(j * _CHUNK, _CHUNK)],
                sem,
            )
            for j in range(n_chunks)
        ]
        for cp in copies:
            cp.wait()
        pltpu.sync_copy(rows_v, out_hbm.at[pl.ds(base, b_per_w)])

    return gather_kernel


def kernel(labels, train, table):
    original_shape = labels.shape
    flat = labels.reshape(-1).astype(jnp.int32)
    key = jax.random.key(42)
    drop_ids = jax.random.uniform(key, flat.shape) < _DROPOUT_PROB
    train_on = jnp.asarray(train) != 0
    flat = jnp.where(
        jnp.logical_and(train_on, drop_ids),
        jnp.full_like(flat, _NUM_CLASSES),
        flat,
    )
    b = flat.shape[0]
    d = table.shape[1]
    idx3 = flat.reshape(_NW, b // (_NW * _CHUNK), _CHUNK)
    out = _make_gather(table.shape[0], d, b)(idx3, table)
    return out.reshape(*original_shape, -1)
